# baseline (speedup 1.0000x reference)
import jax
import jax.numpy as jnp
from jax import lax
from jax.experimental import pallas as pl
from jax.experimental.pallas import tpu as pltpu
from jax.experimental.pallas import tpu_sc as plsc

B = 1024
L = 200
H = 128
OUT_L = L + 1
EPS = 1e-12

NC = 2
NS = 16
NW = NC * NS
NCALLS = 2
B_HALF = B // NCALLS
B_PER_W = B_HALF // NW

G0 = 104
G1 = L - G0
POS_STAGE = 208


def _make_body(b_off):
    def _body(ids_hbm, vec_hbm, emb_hbm, pos_hbm, gam_hbm, bet_hbm, out_hbm,
              ids_v, vecs_v, buf0, buf1, buf2, pos_v, gam_v, bet_v,
              sg0, sg1, sg2, so0, so1, so2):
        wid = lax.axis_index("s") * NC + lax.axis_index("c")
        obase = wid * B_PER_W
        gbase = b_off + obase

        pltpu.sync_copy(pos_hbm.at[pl.ds(0, POS_STAGE)], pos_v)
        pltpu.sync_copy(gam_hbm, gam_v)
        pltpu.sync_copy(bet_hbm, bet_v)
        pltpu.sync_copy(ids_hbm.at[pl.ds(gbase * L, B_PER_W * L)], ids_v)
        pltpu.sync_copy(vec_hbm.at[pl.ds(gbase, B_PER_W)], vecs_v)

        bufs = (buf0, buf1, buf2)
        sgs = (sg0, sg1, sg2)
        sos = (so0, so1, so2)

        gam = [gam_v[pl.ds(16 * k, 16)] for k in range(8)]
        bet = [bet_v[pl.ds(16 * k, 16)] for k in range(8)]

        def ln_row(x):
            s = x[0]
            sq = x[0] * x[0]
            for k in range(1, 8):
                s = s + x[k]
                sq = sq + x[k] * x[k]
            tot = jnp.sum(s)
            tot2 = jnp.sum(sq)
            mean = jnp.full((16,), tot, jnp.float32) * (1.0 / H)
            ex2 = jnp.full((16,), tot2, jnp.float32) * (1.0 / H)
            var = jnp.maximum(ex2 - mean * mean, 0.0) + EPS
            bits = plsc.bitcast(var, jnp.int32)
            y = plsc.bitcast(0x5F3759DF - lax.shift_right_logical(bits, 1),
                             jnp.float32)
            h = 0.5 * var
            for _ in range(2):
                y = y * (1.5 - h * (y * y))
            return [(x[k] - mean) * y * gam[k] + bet[k] for k in range(8)]

        def issue_gather(i, bufm, semm):
            pltpu.async_copy(emb_hbm.at[ids_v.at[pl.ds(i * L, G0)]],
                             bufm.at[pl.ds(1, G0)], semm)
            pltpu.async_copy(emb_hbm.at[ids_v.at[pl.ds(i * L + G0, G1)]],
                             bufm.at[pl.ds(1 + G0, G1)], semm)

        def wait_gather(bufm, semm):
            pltpu.make_async_copy(emb_hbm.at[pl.ds(0, L)],
                                  bufm.at[pl.ds(1, L)], semm).wait()

        def compute(i, bufm):
            x0 = [vecs_v[i, pl.ds(16 * k, 16)] + pos_v[1, pl.ds(16 * k, 16)]
                  for k in range(8)]
            y0 = ln_row(x0)
            for k in range(8):
                bufm[0, pl.ds(16 * k, 16)] = y0[k]

            def row_quad(t, rcarry):
                js = (4 * t + 1, 4 * t + 2, 4 * t + 3, 4 * t + 4)
                xs = [[bufm[j, pl.ds(16 * k, 16)]
                       + pos_v[j + 1, pl.ds(16 * k, 16)]
                       for k in range(8)] for j in js]
                ys = [ln_row(x) for x in xs]
                for j, y in zip(js, ys):
                    for k in range(8):
                        bufm[j, pl.ds(16 * k, 16)] = y[k]
                return rcarry

            lax.fori_loop(0, L // 4, row_quad, 0)

        issue_gather(0, buf0, sg0)
        issue_gather(1, buf1, sg1)

        def k_body(k, carry):
            for m in range(3):
                i = 3 * k + m
                bufm, sgm, som = bufs[m], sgs[m], sos[m]
                nxt = (m + 2) % 3

                @pl.when(i < B_PER_W)
                def _process():
                    wait_gather(bufm, sgm)
                    compute(i, bufm)
                    pltpu.async_copy(bufm, out_hbm.at[obase + i], som)

                    @pl.when(i + 2 < B_PER_W)
                    def _refill():
                        @pl.when(i >= 1)
                        def _drain():
                            pltpu.make_async_copy(
                                bufs[nxt], out_hbm.at[obase + i - 1],
                                sos[nxt]).wait()
                        issue_gather(i + 2, bufs[nxt], sgs[nxt])
            return carry

        lax.fori_loop(0, (B_PER_W + 2) // 3, k_body, 0)
        for i in range(B_PER_W - 3, B_PER_W):
            pltpu.make_async_copy(bufs[i % 3], out_hbm.at[obase + i],
                                  sos[i % 3]).wait()

    return _body


@jax.jit
def kernel(input_ids, vectors, word_emb, pos_emb, ln_gamma, ln_beta):
    ids_flat = input_ids.astype(jnp.int32).reshape(B * L)
    mesh = plsc.VectorSubcoreMesh(core_axis_name="c", subcore_axis_name="s",
                                  num_cores=NC, num_subcores=NS)
    outs = []
    for b_off in range(0, B, B_HALF):
        run = pl.kernel(
            _make_body(b_off),
            out_type=jax.ShapeDtypeStruct((B_HALF, OUT_L, H), jnp.float32),
            mesh=mesh,
            compiler_params=pltpu.CompilerParams(needs_layout_passes=False),
            scratch_types=[
                pltpu.VMEM((B_PER_W * L,), jnp.int32),
                pltpu.VMEM((B_PER_W, H), jnp.float32),
                pltpu.VMEM((OUT_L, H), jnp.float32),
                pltpu.VMEM((OUT_L, H), jnp.float32),
                pltpu.VMEM((OUT_L, H), jnp.float32),
                pltpu.VMEM((POS_STAGE, H), jnp.float32),
                pltpu.VMEM((H,), jnp.float32),
                pltpu.VMEM((H,), jnp.float32),
                pltpu.SemaphoreType.DMA,
                pltpu.SemaphoreType.DMA,
                pltpu.SemaphoreType.DMA,
                pltpu.SemaphoreType.DMA,
                pltpu.SemaphoreType.DMA,
                pltpu.SemaphoreType.DMA,
            ],
        )
        outs.append(run(ids_flat, vectors, word_emb, pos_emb,
                        ln_gamma, ln_beta))
    return jnp.concatenate(outs, axis=0)


# speedup vs baseline: 1.4004x; 1.4004x over previous
import jax
import jax.numpy as jnp
from jax import lax
from jax.experimental import pallas as pl
from jax.experimental.pallas import tpu as pltpu
from jax.experimental.pallas import tpu_sc as plsc

B = 1024
L = 200
H = 128
OUT_L = L + 1
EPS = 1e-12

NC = 2
NS = 16
NW = NC * NS
B_PER_W = B // NW

G0 = 104
G1 = L - G0
POS_STAGE = 208


def _body(ids_hbm, vec_hbm, emb_hbm, pos_hbm, gam_hbm, bet_hbm, out_hbm,
          ids_v, vecs_v, buf0, buf1, buf2, pos_v, gam_v, bet_v,
          sg0, sg1, sg2, so0, so1, so2):
    wid = lax.axis_index("s") * NC + lax.axis_index("c")
    base = wid * B_PER_W

    pltpu.sync_copy(pos_hbm.at[pl.ds(0, POS_STAGE)], pos_v)
    pltpu.sync_copy(gam_hbm, gam_v)
    pltpu.sync_copy(bet_hbm, bet_v)
    pltpu.sync_copy(ids_hbm.at[pl.ds(base * L, B_PER_W * L)], ids_v)
    pltpu.sync_copy(vec_hbm.at[wid], vecs_v)

    bufs = (buf0, buf1, buf2)
    sgs = (sg0, sg1, sg2)
    sos = (so0, so1, so2)

    gam = [gam_v[pl.ds(16 * k, 16)] for k in range(8)]
    bet = [bet_v[pl.ds(16 * k, 16)] for k in range(8)]

    def ln_row(x):
        s = x[0]
        sq = x[0] * x[0]
        for k in range(1, 8):
            s = s + x[k]
            sq = sq + x[k] * x[k]
        tot = jnp.sum(s)
        tot2 = jnp.sum(sq)
        mean = jnp.full((16,), tot, jnp.float32) * (1.0 / H)
        ex2 = jnp.full((16,), tot2, jnp.float32) * (1.0 / H)
        var = jnp.maximum(ex2 - mean * mean, 0.0) + EPS
        bits = plsc.bitcast(var, jnp.int32)
        y = plsc.bitcast(0x5F3759DF - lax.shift_right_logical(bits, 1),
                         jnp.float32)
        h = 0.5 * var
        for _ in range(2):
            y = y * (1.5 - h * (y * y))
        return [(x[k] - mean) * y * gam[k] + bet[k] for k in range(8)]

    def issue_gather(i, bufm, semm):
        pltpu.async_copy(emb_hbm.at[ids_v.at[pl.ds(i * L, G0)]],
                         bufm.at[pl.ds(1, G0)], semm)
        pltpu.async_copy(emb_hbm.at[ids_v.at[pl.ds(i * L + G0, G1)]],
                         bufm.at[pl.ds(1 + G0, G1)], semm)

    def wait_gather(bufm, semm):
        pltpu.make_async_copy(emb_hbm.at[pl.ds(0, L)],
                              bufm.at[pl.ds(1, L)], semm).wait()

    def compute(i, bufm):
        x0 = [vecs_v[i, pl.ds(16 * k, 16)] + pos_v[1, pl.ds(16 * k, 16)]
              for k in range(8)]
        y0 = ln_row(x0)
        for k in range(8):
            bufm[0, pl.ds(16 * k, 16)] = y0[k]

        def row_quad(t, rcarry):
            js = (4 * t + 1, 4 * t + 2, 4 * t + 3, 4 * t + 4)
            xs = [[bufm[j, pl.ds(16 * k, 16)] + pos_v[j + 1, pl.ds(16 * k, 16)]
                   for k in range(8)] for j in js]
            ys = [ln_row(x) for x in xs]
            for j, y in zip(js, ys):
                for k in range(8):
                    bufm[j, pl.ds(16 * k, 16)] = y[k]
            return rcarry

        lax.fori_loop(0, L // 4, row_quad, 0)

    issue_gather(0, buf0, sg0)
    issue_gather(1, buf1, sg1)

    def k_body(k, carry):
        for m in range(3):
            i = 3 * k + m
            bufm, sgm, som = bufs[m], sgs[m], sos[m]
            nxt = (m + 2) % 3

            @pl.when(i < B_PER_W)
            def _process():
                wait_gather(bufm, sgm)
                compute(i, bufm)
                pltpu.async_copy(bufm, out_hbm.at[base + i], som)

                @pl.when(i + 2 < B_PER_W)
                def _refill():
                    @pl.when(i >= 1)
                    def _drain():
                        pltpu.make_async_copy(
                            bufs[nxt], out_hbm.at[base + i - 1],
                            sos[nxt]).wait()
                    issue_gather(i + 2, bufs[nxt], sgs[nxt])
        return carry

    lax.fori_loop(0, (B_PER_W + 2) // 3, k_body, 0)
    pltpu.make_async_copy(buf2, out_hbm.at[base + B_PER_W - 3], so2).wait()
    pltpu.make_async_copy(buf0, out_hbm.at[base + B_PER_W - 2], so0).wait()
    pltpu.make_async_copy(buf1, out_hbm.at[base + B_PER_W - 1], so1).wait()


@jax.jit
def kernel(input_ids, vectors, word_emb, pos_emb, ln_gamma, ln_beta):
    ids_flat = input_ids.astype(jnp.int32).reshape(B * L)
    vec3 = vectors.reshape(NW, B_PER_W, H)
    mesh = plsc.VectorSubcoreMesh(core_axis_name="c", subcore_axis_name="s",
                                  num_cores=NC, num_subcores=NS)
    run = pl.kernel(
        _body,
        out_type=jax.ShapeDtypeStruct((B, OUT_L, H), jnp.float32),
        mesh=mesh,
        compiler_params=pltpu.CompilerParams(needs_layout_passes=False),
        scratch_types=[
            pltpu.VMEM((B_PER_W * L,), jnp.int32),
            pltpu.VMEM((B_PER_W, H), jnp.float32),
            pltpu.VMEM((OUT_L, H), jnp.float32),
            pltpu.VMEM((OUT_L, H), jnp.float32),
            pltpu.VMEM((OUT_L, H), jnp.float32),
            pltpu.VMEM((POS_STAGE, H), jnp.float32),
            pltpu.VMEM((H,), jnp.float32),
            pltpu.VMEM((H,), jnp.float32),
            pltpu.SemaphoreType.DMA,
            pltpu.SemaphoreType.DMA,
            pltpu.SemaphoreType.DMA,
            pltpu.SemaphoreType.DMA,
            pltpu.SemaphoreType.DMA,
            pltpu.SemaphoreType.DMA,
        ],
    )
    return run(ids_flat, vec3, word_emb, pos_emb, ln_gamma, ln_beta)
